# baseline (speedup 1.0000x reference)
import functools

import jax
import jax.numpy as jnp
import numpy as np
from jax import lax
from jax.experimental import pallas as pl
from jax.experimental.pallas import tpu as pltpu
from jax.experimental.pallas import tpu_sc as plsc

N = 10000
E = 320000
D = 128
L = 16
NC = 2
NS = 16
NW = NC * NS
CHUNK = 64
NB = 4
CPW0 = 160
CPW1 = 160
EPAD = NS * (CPW0 + CPW1) * CHUNK
NPAD = 10112
ROWS_PER_TILE = NPAD // NS
MM_BLOCK = 2000

_g = np.arange(D) // 32
_o = np.arange(D) % 32
_COLPERM = _g * 32 + np.where(_o % 2 == 0, _o // 2, 16 + _o // 2)


def _mm_body(x_ref, k_ref, sk_ref, b_ref, h_ref, z_ref):
    xb = x_ref[...]
    h_ref[...] = jnp.dot(
        xb, k_ref[...], preferred_element_type=jnp.float32
    ).astype(jnp.bfloat16)
    z_ref[...] = (
        jnp.dot(xb, sk_ref[...], preferred_element_type=jnp.float32)
        + b_ref[...]
    )


def _matmuls(x2d, w, sw, bias):
    grid = N // MM_BLOCK
    return pl.pallas_call(
        _mm_body,
        grid=(grid,),
        in_specs=[
            pl.BlockSpec((MM_BLOCK, D), lambda i: (i, 0)),
            pl.BlockSpec((D, D), lambda i: (0, 0)),
            pl.BlockSpec((D, D), lambda i: (0, 0)),
            pl.BlockSpec((D,), lambda i: (0,)),
        ],
        out_specs=[
            pl.BlockSpec((MM_BLOCK, D), lambda i: (i, 0)),
            pl.BlockSpec((MM_BLOCK, D), lambda i: (i, 0)),
        ],
        out_shape=[
            jax.ShapeDtypeStruct((N, D), jnp.bfloat16),
            jax.ShapeDtypeStruct((N, D), jnp.float32),
        ],
    )(x2d, w, sw, bias)


def _fin_body(z_ref, p0_ref, p1_ref, o_ref):
    o_ref[...] = jnp.maximum(z_ref[...] + p0_ref[...] + p1_ref[...], 0.0)


def _finalize(z, p0, p1):
    grid = N // MM_BLOCK
    spec = pl.BlockSpec((MM_BLOCK, D), lambda i: (i, 0))
    return pl.pallas_call(
        _fin_body,
        grid=(grid,),
        in_specs=[spec, spec, spec],
        out_specs=spec,
        out_shape=jax.ShapeDtypeStruct((N, D), jnp.float32),
    )(z, p0, p1)


def _sc_aggregate_body(h_hbm, src_hbm, dst_hbm, ew_hbm, p0_hbm, p1_hbm,
                       six, dix, ewr, grow, msg, acc,
                       gsem, ssem, isem, dsem, esem):
    c = lax.axis_index("c")
    s = lax.axis_index("s")
    cpw = jnp.where(c == 0, CPW0, CPW1)
    wbase = jnp.where(c == 0, s * CPW0, NS * CPW0 + s * CPW1) * CHUNK


    def _issue_src_at(b, j):
        pltpu.async_copy(
            src_hbm.at[pl.ds(wbase + j * CHUNK, CHUNK)], six.at[b], isem[b]
        )

    def _issue_dst_ew_at(b, j):
        pltpu.async_copy(
            dst_hbm.at[pl.ds(wbase + j * CHUNK, CHUNK)], dix.at[b], dsem[b]
        )
        pltpu.async_copy(
            ew_hbm.at[pl.ds(wbase + j * CHUNK, CHUNK)],
            ewr.at[pl.ds(b * CHUNK, CHUNK)], esem[b],
        )

    def _wait_idx(semx, b):
        pltpu.make_async_copy(
            src_hbm.at[pl.ds(0, CHUNK)], six.at[b], semx
        ).wait()

    def _wait_ew(b):
        pltpu.make_async_copy(
            ew_hbm.at[pl.ds(0, CHUNK)],
            ewr.at[pl.ds(b * CHUNK, CHUNK)], esem[b],
        ).wait()

    def _wait_gather(b):
        pltpu.make_async_copy(
            h_hbm.at[pl.ds(0, CHUNK)], grow.at[b], gsem[b]
        ).wait()

    def _wait_scatter(p):
        pltpu.make_async_copy(
            p0_hbm.at[pl.ds(0, CHUNK)], msg.at[p], ssem[p]
        ).wait()

    def _scale(b, p):
        def _row(r, rcarry):
            w = plsc.load_gather(
                ewr, [jnp.full((L,), b * CHUNK + r, jnp.int32)]
            )
            for g in range(D // 32):
                v = grow[b, r, pl.ds(g * L, L)]
                lo = plsc.bitcast(v << 16, jnp.float32)
                hi = plsc.bitcast(v & jnp.int32(-65536), jnp.float32)
                msg[p, r, pl.ds(g * 32, L)] = lo * w
                msg[p, r, pl.ds(g * 32 + L, L)] = hi * w
            return rcarry

        lax.fori_loop(0, CHUNK, _row, 0)

    for b in range(NB):
        _issue_src_at(b, b)
    for b in range(2):
        _issue_dst_ew_at(b, b)
    for b in range(3):
        _wait_idx(isem[b], b)
        pltpu.async_copy(h_hbm.at[six.at[b]], grow.at[b], gsem[b])

    def _zrow(r, carry):
        for f in range(D // L):
            msg[0, r, pl.ds(f * L, L)] = jnp.zeros((L,), jnp.float32)
        return carry

    lax.fori_loop(0, CHUNK, _zrow, 0)
    for k in range(ROWS_PER_TILE // CHUNK):
        pltpu.sync_copy(
            msg.at[0],
            acc.at[pl.ds(s * ROWS_PER_TILE + k * CHUNK, CHUNK)],
        )
    _tail = ROWS_PER_TILE % CHUNK
    if _tail:
        pltpu.sync_copy(
            msg.at[0, pl.ds(0, _tail)],
            acc.at[pl.ds(s * ROWS_PER_TILE + ROWS_PER_TILE - _tail, _tail)],
        )
    plsc.subcore_barrier()

    def _slots(jo, carry):
        for b in range(NB):
            j = NB * jo + b
            bn = (b + 2) % NB
            b3 = (b + 3) % NB
            mp = b % 2
            _wait_gather(b)

            @pl.when(j >= 2)
            def _():
                _wait_scatter(mp)

            @pl.when(j + 2 < cpw)
            def _():
                _issue_dst_ew_at(bn, j + 2)

            @pl.when(j + 3 < cpw)
            def _():
                _wait_idx(isem[b3], b3)
                pltpu.async_copy(
                    h_hbm.at[six.at[b3]], grow.at[b3], gsem[b3]
                )

            @pl.when(j + NB < cpw)
            def _():
                _issue_src_at(b, j + NB)

            _wait_ew(b)
            _scale(b, mp)
            _wait_idx(dsem[b], b)
            pltpu.async_copy(
                msg.at[mp], acc.at[dix.at[b]], ssem[mp], add=True
            )
        return carry

    lax.fori_loop(0, cpw // NB, _slots, 0)
    _wait_scatter(0)
    _wait_scatter(1)
    plsc.subcore_barrier()

    @pl.when(c == 0)
    def _():
        pltpu.sync_copy(
            acc.at[pl.ds(s * ROWS_PER_TILE, ROWS_PER_TILE)],
            p0_hbm.at[pl.ds(s * ROWS_PER_TILE, ROWS_PER_TILE)],
        )

    @pl.when(c == 1)
    def _():
        pltpu.sync_copy(
            acc.at[pl.ds(s * ROWS_PER_TILE, ROWS_PER_TILE)],
            p1_hbm.at[pl.ds(s * ROWS_PER_TILE, ROWS_PER_TILE)],
        )


@functools.cache
def _sc_aggregate():
    return pl.kernel(
        _sc_aggregate_body,
        out_type=(
            jax.ShapeDtypeStruct((NPAD, D), jnp.float32),
            jax.ShapeDtypeStruct((NPAD, D), jnp.float32),
        ),
        mesh=plsc.VectorSubcoreMesh(
            core_axis_name="c", subcore_axis_name="s",
            num_cores=NC, num_subcores=NS,
        ),
        scratch_types=[
            pltpu.VMEM((NB, CHUNK), jnp.int32),
            pltpu.VMEM((NB, CHUNK), jnp.int32),
            pltpu.VMEM((NB * CHUNK,), jnp.float32),
            pltpu.VMEM((NB, CHUNK, D // 2), jnp.int32),
            pltpu.VMEM((2, CHUNK, D), jnp.float32),
            pltpu.VMEM_SHARED((NPAD, D), jnp.float32),
            [pltpu.SemaphoreType.DMA] * NB,
            [pltpu.SemaphoreType.DMA] * 2,
            [pltpu.SemaphoreType.DMA] * NB,
            [pltpu.SemaphoreType.DMA] * NB,
            [pltpu.SemaphoreType.DMA] * NB,
        ],
        compiler_params=pltpu.CompilerParams(
            needs_layout_passes=False, use_tc_tiling_on_sc=False
        ),
    )


def kernel(x, edge_index, edge_weight, kernel, self_kernel,
           self_loop_weight, bias):
    x2d = jnp.squeeze(x, axis=0)
    sk_scaled = self_kernel * self_loop_weight[0]
    h, z = _matmuls(x2d, kernel[:, _COLPERM], sk_scaled, bias)
    h32 = jax.lax.bitcast_convert_type(
        h.reshape(N, D // 2, 2), jnp.int32
    )

    pad = EPAD - E
    src = jnp.concatenate(
        [edge_index[0].astype(jnp.int32), jnp.zeros((pad,), jnp.int32)]
    )
    dst = jnp.concatenate(
        [edge_index[1].astype(jnp.int32), jnp.zeros((pad,), jnp.int32)]
    )
    ew = jnp.concatenate(
        [edge_weight.astype(jnp.float32), jnp.zeros((pad,), jnp.float32)]
    )

    p0, p1 = _sc_aggregate()(h32, src, dst, ew)
    out = _finalize(z, p0, p1)
    return out[None, :, :]


# speedup vs baseline: 1.0068x; 1.0068x over previous
import functools

import jax
import jax.numpy as jnp
import numpy as np
from jax import lax
from jax.experimental import pallas as pl
from jax.experimental.pallas import tpu as pltpu
from jax.experimental.pallas import tpu_sc as plsc

N = 10000
E = 320000
D = 128
L = 16
NC = 2
NS = 16
NW = NC * NS
CHUNK = 64
NB = 4
CPW0 = 160
CPW1 = 160
EPAD = NS * (CPW0 + CPW1) * CHUNK
NPAD = 10112
ROWS_PER_TILE = NPAD // NS
MM_BLOCK = 2000

_g = np.arange(D) // 32
_o = np.arange(D) % 32
_COLPERM = _g * 32 + np.where(_o % 2 == 0, _o // 2, 16 + _o // 2)


def _mm_body(x_ref, k_ref, sk_ref, b_ref, h_ref, z_ref):
    xb = x_ref[...]
    h_ref[...] = jnp.dot(
        xb, k_ref[...], preferred_element_type=jnp.float32
    ).astype(jnp.bfloat16)
    z_ref[...] = (
        jnp.dot(xb, sk_ref[...], preferred_element_type=jnp.float32)
        + b_ref[...]
    )


def _matmuls(x2d, w, sw, bias):
    grid = N // MM_BLOCK
    return pl.pallas_call(
        _mm_body,
        grid=(grid,),
        in_specs=[
            pl.BlockSpec((MM_BLOCK, D), lambda i: (i, 0)),
            pl.BlockSpec((D, D), lambda i: (0, 0)),
            pl.BlockSpec((D, D), lambda i: (0, 0)),
            pl.BlockSpec((D,), lambda i: (0,)),
        ],
        out_specs=[
            pl.BlockSpec((MM_BLOCK, D), lambda i: (i, 0)),
            pl.BlockSpec((MM_BLOCK, D), lambda i: (i, 0)),
        ],
        out_shape=[
            jax.ShapeDtypeStruct((N, D), jnp.bfloat16),
            jax.ShapeDtypeStruct((N, D), jnp.float32),
        ],
    )(x2d, w, sw, bias)


def _fin_body(z_ref, p0_ref, p1_ref, o_ref):
    o_ref[...] = jnp.maximum(z_ref[...] + p0_ref[...] + p1_ref[...], 0.0)


def _finalize(z, p0, p1):
    grid = N // MM_BLOCK
    spec = pl.BlockSpec((MM_BLOCK, D), lambda i: (i, 0))
    return pl.pallas_call(
        _fin_body,
        grid=(grid,),
        in_specs=[spec, spec, spec],
        out_specs=spec,
        out_shape=jax.ShapeDtypeStruct((N, D), jnp.float32),
    )(z, p0, p1)


def _sc_aggregate_body(h_hbm, src_hbm, dst_hbm, ew_hbm, p0_hbm, p1_hbm,
                       six, dix, ewr, grow, msg, acc,
                       gsem, ssem, isem, dsem, esem):
    c = lax.axis_index("c")
    s = lax.axis_index("s")
    cpw = jnp.where(c == 0, CPW0, CPW1)
    wbase = jnp.where(c == 0, s * CPW0, NS * CPW0 + s * CPW1) * CHUNK


    def _issue_src_at(b, j):
        pltpu.async_copy(
            src_hbm.at[pl.ds(wbase + j * CHUNK, CHUNK)], six.at[b], isem[b]
        )

    def _issue_dst_ew_at(b, j):
        pltpu.async_copy(
            dst_hbm.at[pl.ds(wbase + j * CHUNK, CHUNK)], dix.at[b], dsem[b]
        )
        pltpu.async_copy(
            ew_hbm.at[pl.ds(wbase + j * CHUNK, CHUNK)],
            ewr.at[pl.ds(b * CHUNK, CHUNK)], esem[b],
        )

    def _wait_idx(semx, b):
        pltpu.make_async_copy(
            src_hbm.at[pl.ds(0, CHUNK)], six.at[b], semx
        ).wait()

    def _wait_ew(b):
        pltpu.make_async_copy(
            ew_hbm.at[pl.ds(0, CHUNK)],
            ewr.at[pl.ds(b * CHUNK, CHUNK)], esem[b],
        ).wait()

    def _wait_gather(b):
        pltpu.make_async_copy(
            h_hbm.at[pl.ds(0, CHUNK)], grow.at[b], gsem[b]
        ).wait()

    def _wait_scatter(p):
        pltpu.make_async_copy(
            p0_hbm.at[pl.ds(0, CHUNK)], msg.at[p], ssem[p]
        ).wait()

    def _scale(b, p):
        def _rows(ri, rcarry):
            r0 = ri * 2
            for r in (r0, r0 + 1):
                w = plsc.load_gather(
                    ewr, [jnp.full((L,), b * CHUNK + r, jnp.int32)]
                )
                for g in range(D // 32):
                    v = grow[b, r, pl.ds(g * L, L)]
                    lo = plsc.bitcast(v << 16, jnp.float32)
                    hi = plsc.bitcast(v & jnp.int32(-65536), jnp.float32)
                    msg[p, r, pl.ds(g * 32, L)] = lo * w
                    msg[p, r, pl.ds(g * 32 + L, L)] = hi * w
            return rcarry

        lax.fori_loop(0, CHUNK // 2, _rows, 0)

    for b in range(NB):
        _issue_src_at(b, b)
    for b in range(2):
        _issue_dst_ew_at(b, b)
    for b in range(3):
        _wait_idx(isem[b], b)
        pltpu.async_copy(h_hbm.at[six.at[b]], grow.at[b], gsem[b])

    def _zrow(r, carry):
        for f in range(D // L):
            msg[0, r, pl.ds(f * L, L)] = jnp.zeros((L,), jnp.float32)
        return carry

    lax.fori_loop(0, CHUNK, _zrow, 0)
    for k in range(ROWS_PER_TILE // CHUNK):
        pltpu.sync_copy(
            msg.at[0],
            acc.at[pl.ds(s * ROWS_PER_TILE + k * CHUNK, CHUNK)],
        )
    _tail = ROWS_PER_TILE % CHUNK
    if _tail:
        pltpu.sync_copy(
            msg.at[0, pl.ds(0, _tail)],
            acc.at[pl.ds(s * ROWS_PER_TILE + ROWS_PER_TILE - _tail, _tail)],
        )
    plsc.subcore_barrier()

    def _slots(jo, carry):
        for b in range(NB):
            j = NB * jo + b
            bn = (b + 2) % NB
            b3 = (b + 3) % NB
            mp = b % 2
            _wait_gather(b)

            @pl.when(j >= 2)
            def _():
                _wait_scatter(mp)

            @pl.when(j + 2 < cpw)
            def _():
                _issue_dst_ew_at(bn, j + 2)

            @pl.when(j + 3 < cpw)
            def _():
                _wait_idx(isem[b3], b3)
                pltpu.async_copy(
                    h_hbm.at[six.at[b3]], grow.at[b3], gsem[b3]
                )

            @pl.when(j + NB < cpw)
            def _():
                _issue_src_at(b, j + NB)

            _wait_ew(b)
            _scale(b, mp)
            _wait_idx(dsem[b], b)
            pltpu.async_copy(
                msg.at[mp], acc.at[dix.at[b]], ssem[mp], add=True
            )
        return carry

    lax.fori_loop(0, cpw // NB, _slots, 0)
    _wait_scatter(0)
    _wait_scatter(1)
    plsc.subcore_barrier()

    @pl.when(c == 0)
    def _():
        pltpu.sync_copy(
            acc.at[pl.ds(s * ROWS_PER_TILE, ROWS_PER_TILE)],
            p0_hbm.at[pl.ds(s * ROWS_PER_TILE, ROWS_PER_TILE)],
        )

    @pl.when(c == 1)
    def _():
        pltpu.sync_copy(
            acc.at[pl.ds(s * ROWS_PER_TILE, ROWS_PER_TILE)],
            p1_hbm.at[pl.ds(s * ROWS_PER_TILE, ROWS_PER_TILE)],
        )


@functools.cache
def _sc_aggregate():
    return pl.kernel(
        _sc_aggregate_body,
        out_type=(
            jax.ShapeDtypeStruct((NPAD, D), jnp.float32),
            jax.ShapeDtypeStruct((NPAD, D), jnp.float32),
        ),
        mesh=plsc.VectorSubcoreMesh(
            core_axis_name="c", subcore_axis_name="s",
            num_cores=NC, num_subcores=NS,
        ),
        scratch_types=[
            pltpu.VMEM((NB, CHUNK), jnp.int32),
            pltpu.VMEM((NB, CHUNK), jnp.int32),
            pltpu.VMEM((NB * CHUNK,), jnp.float32),
            pltpu.VMEM((NB, CHUNK, D // 2), jnp.int32),
            pltpu.VMEM((2, CHUNK, D), jnp.float32),
            pltpu.VMEM_SHARED((NPAD, D), jnp.float32),
            [pltpu.SemaphoreType.DMA] * NB,
            [pltpu.SemaphoreType.DMA] * 2,
            [pltpu.SemaphoreType.DMA] * NB,
            [pltpu.SemaphoreType.DMA] * NB,
            [pltpu.SemaphoreType.DMA] * NB,
        ],
        compiler_params=pltpu.CompilerParams(
            needs_layout_passes=False, use_tc_tiling_on_sc=False
        ),
    )


def kernel(x, edge_index, edge_weight, kernel, self_kernel,
           self_loop_weight, bias):
    x2d = jnp.squeeze(x, axis=0)
    sk_scaled = self_kernel * self_loop_weight[0]
    h, z = _matmuls(x2d, kernel[:, _COLPERM], sk_scaled, bias)
    h32 = jax.lax.bitcast_convert_type(
        h.reshape(N, D // 2, 2), jnp.int32
    )

    pad = EPAD - E
    src = jnp.concatenate(
        [edge_index[0].astype(jnp.int32), jnp.zeros((pad,), jnp.int32)]
    )
    dst = jnp.concatenate(
        [edge_index[1].astype(jnp.int32), jnp.zeros((pad,), jnp.int32)]
    )
    ew = jnp.concatenate(
        [edge_weight.astype(jnp.float32), jnp.zeros((pad,), jnp.float32)]
    )

    p0, p1 = _sc_aggregate()(h32, src, dst, ew)
    out = _finalize(z, p0, p1)
    return out[None, :, :]


# speedup vs baseline: 1.0086x; 1.0018x over previous
import functools

import jax
import jax.numpy as jnp
import numpy as np
from jax import lax
from jax.experimental import pallas as pl
from jax.experimental.pallas import tpu as pltpu
from jax.experimental.pallas import tpu_sc as plsc

N = 10000
E = 320000
D = 128
L = 16
NC = 2
NS = 16
NW = NC * NS
CHUNK = 80
NB = 4
CPW0 = 128
CPW1 = 128
EPAD = NS * (CPW0 + CPW1) * CHUNK
NPAD = 10112
ROWS_PER_TILE = NPAD // NS
MM_BLOCK = 2000

_g = np.arange(D) // 32
_o = np.arange(D) % 32
_COLPERM = _g * 32 + np.where(_o % 2 == 0, _o // 2, 16 + _o // 2)


def _mm_body(x_ref, k_ref, sk_ref, b_ref, h_ref, z_ref):
    xb = x_ref[...]
    h_ref[...] = jnp.dot(
        xb, k_ref[...], preferred_element_type=jnp.float32
    ).astype(jnp.bfloat16)
    z_ref[...] = (
        jnp.dot(xb, sk_ref[...], preferred_element_type=jnp.float32)
        + b_ref[...]
    )


def _matmuls(x2d, w, sw, bias):
    grid = N // MM_BLOCK
    return pl.pallas_call(
        _mm_body,
        grid=(grid,),
        in_specs=[
            pl.BlockSpec((MM_BLOCK, D), lambda i: (i, 0)),
            pl.BlockSpec((D, D), lambda i: (0, 0)),
            pl.BlockSpec((D, D), lambda i: (0, 0)),
            pl.BlockSpec((D,), lambda i: (0,)),
        ],
        out_specs=[
            pl.BlockSpec((MM_BLOCK, D), lambda i: (i, 0)),
            pl.BlockSpec((MM_BLOCK, D), lambda i: (i, 0)),
        ],
        out_shape=[
            jax.ShapeDtypeStruct((N, D), jnp.bfloat16),
            jax.ShapeDtypeStruct((N, D), jnp.float32),
        ],
    )(x2d, w, sw, bias)


def _fin_body(z_ref, p0_ref, p1_ref, o_ref):
    o_ref[...] = jnp.maximum(z_ref[...] + p0_ref[...] + p1_ref[...], 0.0)


def _finalize(z, p0, p1):
    grid = N // MM_BLOCK
    spec = pl.BlockSpec((MM_BLOCK, D), lambda i: (i, 0))
    return pl.pallas_call(
        _fin_body,
        grid=(grid,),
        in_specs=[spec, spec, spec],
        out_specs=spec,
        out_shape=jax.ShapeDtypeStruct((N, D), jnp.float32),
    )(z, p0, p1)


def _sc_aggregate_body(h_hbm, src_hbm, dst_hbm, ew_hbm, p0_hbm, p1_hbm,
                       six, dix, ewr, grow, msg, acc,
                       gsem, ssem, isem, dsem, esem):
    c = lax.axis_index("c")
    s = lax.axis_index("s")
    cpw = jnp.where(c == 0, CPW0, CPW1)
    wbase = jnp.where(c == 0, s * CPW0, NS * CPW0 + s * CPW1) * CHUNK


    def _issue_src_at(b, j):
        pltpu.async_copy(
            src_hbm.at[pl.ds(wbase + j * CHUNK, CHUNK)], six.at[b], isem[b]
        )

    def _issue_dst_ew_at(b, j):
        pltpu.async_copy(
            dst_hbm.at[pl.ds(wbase + j * CHUNK, CHUNK)], dix.at[b], dsem[b]
        )
        pltpu.async_copy(
            ew_hbm.at[pl.ds(wbase + j * CHUNK, CHUNK)],
            ewr.at[pl.ds(b * CHUNK, CHUNK)], esem[b],
        )

    def _wait_idx(semx, b):
        pltpu.make_async_copy(
            src_hbm.at[pl.ds(0, CHUNK)], six.at[b], semx
        ).wait()

    def _wait_ew(b):
        pltpu.make_async_copy(
            ew_hbm.at[pl.ds(0, CHUNK)],
            ewr.at[pl.ds(b * CHUNK, CHUNK)], esem[b],
        ).wait()

    def _wait_gather(b):
        pltpu.make_async_copy(
            h_hbm.at[pl.ds(0, CHUNK)], grow.at[b], gsem[b]
        ).wait()

    def _wait_scatter(p):
        pltpu.make_async_copy(
            p0_hbm.at[pl.ds(0, CHUNK)], msg.at[p], ssem[p]
        ).wait()

    def _scale(b, p):
        def _rows(ri, rcarry):
            r0 = ri * 2
            for r in (r0, r0 + 1):
                w = plsc.load_gather(
                    ewr, [jnp.full((L,), b * CHUNK + r, jnp.int32)]
                )
                for g in range(D // 32):
                    v = grow[b, r, pl.ds(g * L, L)]
                    lo = plsc.bitcast(v << 16, jnp.float32)
                    hi = plsc.bitcast(v & jnp.int32(-65536), jnp.float32)
                    msg[p, r, pl.ds(g * 32, L)] = lo * w
                    msg[p, r, pl.ds(g * 32 + L, L)] = hi * w
            return rcarry

        lax.fori_loop(0, CHUNK // 2, _rows, 0)

    for b in range(NB):
        _issue_src_at(b, b)
    for b in range(2):
        _issue_dst_ew_at(b, b)
    for b in range(3):
        _wait_idx(isem[b], b)
        pltpu.async_copy(h_hbm.at[six.at[b]], grow.at[b], gsem[b])

    def _zrow(r, carry):
        for f in range(D // L):
            msg[0, r, pl.ds(f * L, L)] = jnp.zeros((L,), jnp.float32)
        return carry

    lax.fori_loop(0, CHUNK, _zrow, 0)
    for k in range(ROWS_PER_TILE // CHUNK):
        pltpu.sync_copy(
            msg.at[0],
            acc.at[pl.ds(s * ROWS_PER_TILE + k * CHUNK, CHUNK)],
        )
    _tail = ROWS_PER_TILE % CHUNK
    if _tail:
        pltpu.sync_copy(
            msg.at[0, pl.ds(0, _tail)],
            acc.at[pl.ds(s * ROWS_PER_TILE + ROWS_PER_TILE - _tail, _tail)],
        )
    plsc.subcore_barrier()

    def _slots(jo, carry):
        for b in range(NB):
            j = NB * jo + b
            bn = (b + 2) % NB
            b3 = (b + 3) % NB
            mp = b % 2
            _wait_gather(b)

            @pl.when(j >= 2)
            def _():
                _wait_scatter(mp)

            @pl.when(j + 2 < cpw)
            def _():
                _issue_dst_ew_at(bn, j + 2)

            @pl.when(j + 3 < cpw)
            def _():
                _wait_idx(isem[b3], b3)
                pltpu.async_copy(
                    h_hbm.at[six.at[b3]], grow.at[b3], gsem[b3]
                )

            @pl.when(j + NB < cpw)
            def _():
                _issue_src_at(b, j + NB)

            _wait_ew(b)
            _scale(b, mp)
            _wait_idx(dsem[b], b)
            pltpu.async_copy(
                msg.at[mp], acc.at[dix.at[b]], ssem[mp], add=True
            )
        return carry

    lax.fori_loop(0, cpw // NB, _slots, 0)
    _wait_scatter(0)
    _wait_scatter(1)
    plsc.subcore_barrier()

    @pl.when(c == 0)
    def _():
        pltpu.sync_copy(
            acc.at[pl.ds(s * ROWS_PER_TILE, ROWS_PER_TILE)],
            p0_hbm.at[pl.ds(s * ROWS_PER_TILE, ROWS_PER_TILE)],
        )

    @pl.when(c == 1)
    def _():
        pltpu.sync_copy(
            acc.at[pl.ds(s * ROWS_PER_TILE, ROWS_PER_TILE)],
            p1_hbm.at[pl.ds(s * ROWS_PER_TILE, ROWS_PER_TILE)],
        )


@functools.cache
def _sc_aggregate():
    return pl.kernel(
        _sc_aggregate_body,
        out_type=(
            jax.ShapeDtypeStruct((NPAD, D), jnp.float32),
            jax.ShapeDtypeStruct((NPAD, D), jnp.float32),
        ),
        mesh=plsc.VectorSubcoreMesh(
            core_axis_name="c", subcore_axis_name="s",
            num_cores=NC, num_subcores=NS,
        ),
        scratch_types=[
            pltpu.VMEM((NB, CHUNK), jnp.int32),
            pltpu.VMEM((NB, CHUNK), jnp.int32),
            pltpu.VMEM((NB * CHUNK,), jnp.float32),
            pltpu.VMEM((NB, CHUNK, D // 2), jnp.int32),
            pltpu.VMEM((2, CHUNK, D), jnp.float32),
            pltpu.VMEM_SHARED((NPAD, D), jnp.float32),
            [pltpu.SemaphoreType.DMA] * NB,
            [pltpu.SemaphoreType.DMA] * 2,
            [pltpu.SemaphoreType.DMA] * NB,
            [pltpu.SemaphoreType.DMA] * NB,
            [pltpu.SemaphoreType.DMA] * NB,
        ],
        compiler_params=pltpu.CompilerParams(
            needs_layout_passes=False, use_tc_tiling_on_sc=False
        ),
    )


def kernel(x, edge_index, edge_weight, kernel, self_kernel,
           self_loop_weight, bias):
    x2d = jnp.squeeze(x, axis=0)
    sk_scaled = self_kernel * self_loop_weight[0]
    h, z = _matmuls(x2d, kernel[:, _COLPERM], sk_scaled, bias)
    h32 = jax.lax.bitcast_convert_type(
        h.reshape(N, D // 2, 2), jnp.int32
    )

    pad = EPAD - E
    src = jnp.concatenate(
        [edge_index[0].astype(jnp.int32), jnp.zeros((pad,), jnp.int32)]
    )
    dst = jnp.concatenate(
        [edge_index[1].astype(jnp.int32), jnp.zeros((pad,), jnp.int32)]
    )
    ew = jnp.concatenate(
        [edge_weight.astype(jnp.float32), jnp.zeros((pad,), jnp.float32)]
    )

    p0, p1 = _sc_aggregate()(h32, src, dst, ew)
    out = _finalize(z, p0, p1)
    return out[None, :, :]
